# baseline (speedup 1.0000x reference)
import functools
import math

import jax
import jax.numpy as jnp
import numpy as np
from jax import lax
from jax.experimental import pallas as pl
from jax.experimental.pallas import tpu as pltpu
from jax.experimental.pallas import tpu_sc as plsc

VOCAB = 100000
D = 128
L_TOK = 24
B = 4096
MAX_LEN = 25


def _pe_const(max_len, d_model):
    position = np.arange(0, max_len, dtype=np.float32)[:, None]
    div_term = np.exp(
        np.arange(0, d_model, 2).astype(np.float32) * (-math.log(10000.0) / d_model)
    )
    pe = np.zeros((max_len, d_model), dtype=np.float32)
    pe[:, 0::2] = np.sin(position * div_term)
    pe[:, 1::2] = np.cos(position * div_term)
    return pe


_PE = _pe_const(MAX_LEN, D)

_INFO = plsc.get_sparse_core_info()
_NC = _INFO.num_cores
_NS = _INFO.num_subcores
_NW = _NC * _NS

_SPLITS = 2
_BCALL = B // _SPLITS
_B_PER_W = _BCALL // _NW
_NB = 8
_CHUNKS = _B_PER_W // _NB
_TOK = _NB * L_TOK
_OROWS = _NB * MAX_LEN


def _sc_body(words_hbm, cls_hbm, noun_hbm, ct_hbm, spec_hbm, pe_hbm,
             out_hbm,
             widx_v, cls2_v, base_v, ct_v, spec_v,
             ob0, ob1, ob2,
             sem_n0, sem_n1, sem_n2,
             sem_w0, sem_w1, sem_w2):
    wid = lax.axis_index("s") * _NC + lax.axis_index("c")
    b0w = wid * _B_PER_W
    obs = (ob0, ob1, ob2)
    sems_n = (sem_n0, sem_n1, sem_n2)
    sems_w = (sem_w0, sem_w1, sem_w2)

    def fire_noun(k, m):
        for j in range(_NB):
            pltpu.async_copy(
                noun_hbm.at[widx_v.at[k * _NB + j]],
                obs[m].at[pl.ds(j * MAX_LEN + 1, L_TOK)], sems_n[m])

    def drain_noun(m):
        for j in range(_NB):
            pltpu.make_async_copy(
                noun_hbm.at[pl.ds(0, L_TOK)],
                obs[m].at[pl.ds(j * MAX_LEN + 1, L_TOK)], sems_n[m]).wait()

    def fire_writes(k, m):
        b0 = b0w + k * _NB
        for j in range(_NB):
            pltpu.async_copy(obs[m].at[pl.ds(j * MAX_LEN, MAX_LEN)],
                             out_hbm.at[b0 + j], sems_w[m])

    def drain_writes(m):
        for j in range(_NB):
            pltpu.make_async_copy(obs[m].at[pl.ds(j * MAX_LEN, MAX_LEN)],
                                  out_hbm.at[0], sems_w[m]).wait()

    def assemble(k, m, dv):
        o = obs[m]
        iota16 = lax.broadcasted_iota(jnp.int32, (16,), 0)

        def batch(j, dvc):
            kj = k * _NB + j
            ca = (cls2_v[kj, pl.ds(0, 16)] & 1) > 0
            cb = (cls2_v[kj, pl.ds(8, 16)] & 1) > 0

            def tok4(half, lane_off):
                def step(s, dvt):
                    for u in range(4):
                        l = s * 4 + u
                        oh = iota16 == (l - lane_off)
                        cnt = plsc.all_reduce_population_count(half & oh)
                        cvf = cnt.astype(jnp.float32)
                        orow = j * MAX_LEN + 1 + l
                        for q in range(D // 16):
                            sl = pl.ds(q * 16, 16)
                            plsc.addupdate(o.at[orow, sl],
                                           base_v[l, sl] + cvf * dvt[q])
                    return dvt
                return step

            dvc = lax.fori_loop(0, 4, tok4(ca, 0), dvc)
            dvc = lax.fori_loop(4, 6, tok4(cb, 8), dvc)
            return dvc

        return plsc.parallel_loop(0, _NB, carry=dv)(batch)

    def process(k, m, has_next2, drain_w, dv):
        drain_noun(m)
        dv = assemble(k, m, dv)
        pl.when(drain_w)(lambda: drain_writes((m + 2) % 3))
        if has_next2 is not None:
            pl.when(has_next2)(lambda: fire_noun(k + 2, (m + 2) % 3))
        fire_writes(k, m)
        return dv

    pltpu.sync_copy(words_hbm.at[pl.ds(b0w, _B_PER_W)], widx_v)
    pltpu.sync_copy(cls_hbm.at[pl.ds(b0w, _B_PER_W)], cls2_v)
    pltpu.sync_copy(pe_hbm, base_v)
    pltpu.sync_copy(ct_hbm, ct_v)
    pltpu.sync_copy(spec_hbm, spec_v)
    dv = []
    for q in range(D // 16):
        sl = pl.ds(q * 16, 16)
        c0 = ct_v[0, sl]
        dv.append(ct_v[1, sl] - c0)
        for l in range(L_TOK):
            base_v[l, sl] = base_v[l, sl] + c0
    dv = tuple(dv)
    for ov in obs:
        for j in range(_NB):
            for q in range(D // 16):
                sl = pl.ds(q * 16, 16)
                ov[j * MAX_LEN, sl] = spec_v[0, sl]
    fire_noun(0, 0)
    fire_noun(1, 1)

    true_ = jnp.bool_(True)

    n_loop = _CHUNKS // 3 - (1 if _CHUNKS % 3 == 0 else 0)

    def triple(i, dvc):
        k = 3 * i
        dvc = process(k + 0, 0, true_, k >= 1, dvc)
        dvc = process(k + 1, 1, true_, true_, dvc)
        dvc = process(k + 2, 2, 3 * i + 4 < _CHUNKS, true_, dvc)
        return dvc

    dv = lax.fori_loop(0, n_loop, triple, dv)
    for k in range(3 * n_loop, _CHUNKS):
        dv = process(k, k % 3,
                     true_ if k + 2 < _CHUNKS else None,
                     true_ if k >= 1 else jnp.bool_(False), dv)
    drain_writes((_CHUNKS - 1) % 3)


def kernel(words, classes, noun_table, class_table, special_emb):
    words_i = words.astype(jnp.int32)
    cls_i = classes.astype(jnp.int32)
    pe = jnp.asarray(_PE[:L_TOK])

    mesh = plsc.VectorSubcoreMesh(core_axis_name="c", subcore_axis_name="s")
    run = functools.partial(
        pl.kernel,
        mesh=mesh,
        compiler_params=pltpu.CompilerParams(needs_layout_passes=False),
        out_type=jax.ShapeDtypeStruct((_BCALL, MAX_LEN, D), jnp.float32),
        scratch_types=[
            pltpu.VMEM((_B_PER_W, L_TOK), jnp.int32),
            pltpu.VMEM((_B_PER_W, L_TOK), jnp.int32),
            pltpu.VMEM((L_TOK, D), jnp.float32),
            pltpu.VMEM((2, D), jnp.float32),
            pltpu.VMEM((1, D), jnp.float32),
            pltpu.VMEM((_OROWS, D), jnp.float32),
            pltpu.VMEM((_OROWS, D), jnp.float32),
            pltpu.VMEM((_OROWS, D), jnp.float32),
        ] + [pltpu.SemaphoreType.DMA] * 6,
    )(_sc_body)
    parts = [
        run(words_i[s * _BCALL:(s + 1) * _BCALL],
            cls_i[s * _BCALL:(s + 1) * _BCALL],
            noun_table, class_table, special_emb, pe)
        for s in range(_SPLITS)
    ]
    return jnp.concatenate(parts, axis=0)


# speedup vs baseline: 1.3305x; 1.3305x over previous
import functools
import math

import jax
import jax.numpy as jnp
import numpy as np
from jax import lax
from jax.experimental import pallas as pl
from jax.experimental.pallas import tpu as pltpu
from jax.experimental.pallas import tpu_sc as plsc

VOCAB = 100000
D = 128
L_TOK = 24
B = 4096
MAX_LEN = 25


def _pe_const(max_len, d_model):
    position = np.arange(0, max_len, dtype=np.float32)[:, None]
    div_term = np.exp(
        np.arange(0, d_model, 2).astype(np.float32) * (-math.log(10000.0) / d_model)
    )
    pe = np.zeros((max_len, d_model), dtype=np.float32)
    pe[:, 0::2] = np.sin(position * div_term)
    pe[:, 1::2] = np.cos(position * div_term)
    return pe


_PE = _pe_const(MAX_LEN, D)

_INFO = plsc.get_sparse_core_info()
_NC = _INFO.num_cores
_NS = _INFO.num_subcores
_NW = _NC * _NS

_B_PER_W = B // _NW
_NB = 8
_CHUNKS = _B_PER_W // _NB
_TOK = _NB * L_TOK
_OROWS = _NB * MAX_LEN


def _sc_body(words_hbm, cls_hbm, noun_hbm, ct_hbm, spec_hbm, pe_hbm,
             out_hbm,
             widx_v, cls2_v, base_v, ct_v, spec_v,
             ob0, ob1, ob2,
             sem_n0, sem_n1, sem_n2,
             sem_w0, sem_w1, sem_w2):
    wid = lax.axis_index("s") * _NC + lax.axis_index("c")
    b0w = wid * _B_PER_W
    obs = (ob0, ob1, ob2)
    sems_n = (sem_n0, sem_n1, sem_n2)
    sems_w = (sem_w0, sem_w1, sem_w2)

    def fire_noun(k, m):
        for j in range(_NB):
            pltpu.async_copy(
                noun_hbm.at[widx_v.at[k * _NB + j]],
                obs[m].at[pl.ds(j * MAX_LEN + 1, L_TOK)], sems_n[m])

    def drain_noun(m):
        for j in range(_NB):
            pltpu.make_async_copy(
                noun_hbm.at[pl.ds(0, L_TOK)],
                obs[m].at[pl.ds(j * MAX_LEN + 1, L_TOK)], sems_n[m]).wait()

    def fire_writes(k, m):
        b0 = b0w + k * _NB
        for j in range(_NB):
            pltpu.async_copy(obs[m].at[pl.ds(j * MAX_LEN, MAX_LEN)],
                             out_hbm.at[b0 + j], sems_w[m])

    def drain_writes(m):
        for j in range(_NB):
            pltpu.make_async_copy(obs[m].at[pl.ds(j * MAX_LEN, MAX_LEN)],
                                  out_hbm.at[0], sems_w[m]).wait()

    def assemble(k, m, dv):
        o = obs[m]
        iota16 = lax.broadcasted_iota(jnp.int32, (16,), 0)

        def batch(j, dvc):
            kj = k * _NB + j
            ca = (cls2_v[kj, pl.ds(0, 16)] & 1) > 0
            cb = (cls2_v[kj, pl.ds(8, 16)] & 1) > 0

            def tok4(half, lane_off):
                def step(s, dvt):
                    for u in range(4):
                        l = s * 4 + u
                        oh = iota16 == (l - lane_off)
                        cnt = plsc.all_reduce_population_count(half & oh)
                        cvf = cnt.astype(jnp.float32)
                        orow = j * MAX_LEN + 1 + l
                        for q in range(D // 16):
                            sl = pl.ds(q * 16, 16)
                            plsc.addupdate(o.at[orow, sl],
                                           base_v[l, sl] + cvf * dvt[q])
                    return dvt
                return step

            dvc = lax.fori_loop(0, 4, tok4(ca, 0), dvc)
            dvc = lax.fori_loop(4, 6, tok4(cb, 8), dvc)
            return dvc

        return plsc.parallel_loop(0, _NB, carry=dv)(batch)

    def process(k, m, has_next2, drain_w, dv):
        with jax.named_scope("p_drain_noun"):
            drain_noun(m)
        with jax.named_scope("p_assemble"):
            dv = assemble(k, m, dv)
        with jax.named_scope("p_drain_writes"):
            pl.when(drain_w)(lambda: drain_writes((m + 2) % 3))
        with jax.named_scope("p_fire"):
            pl.when(has_next2)(lambda: fire_noun(k + 2, (m + 2) % 3))
            fire_writes(k, m)
        return dv

    pltpu.sync_copy(words_hbm.at[pl.ds(b0w, _B_PER_W)], widx_v)
    pltpu.sync_copy(cls_hbm.at[pl.ds(b0w, _B_PER_W)], cls2_v)
    pltpu.sync_copy(pe_hbm, base_v)
    pltpu.sync_copy(ct_hbm, ct_v)
    pltpu.sync_copy(spec_hbm, spec_v)
    dv = []
    for q in range(D // 16):
        sl = pl.ds(q * 16, 16)
        c0 = ct_v[0, sl]
        dv.append(ct_v[1, sl] - c0)
        for l in range(L_TOK):
            base_v[l, sl] = base_v[l, sl] + c0
    dv = tuple(dv)
    for ov in obs:
        for j in range(_NB):
            for q in range(D // 16):
                sl = pl.ds(q * 16, 16)
                ov[j * MAX_LEN, sl] = spec_v[0, sl]
    fire_noun(0, 0)
    fire_noun(1, 1)

    true_ = jnp.bool_(True)

    def triple(i, dvc):
        k = 3 * i
        dvc = process(k + 0, 0, true_, k >= 1, dvc)
        dvc = process(k + 1, 1, true_, true_, dvc)
        dvc = process(k + 2, 2, k + 4 < _CHUNKS, true_, dvc)
        return dvc

    dv = lax.fori_loop(0, (_CHUNKS - 1) // 3, triple, dv)
    false_ = jnp.bool_(False)
    process(_CHUNKS - 1, (_CHUNKS - 1) % 3, false_, true_, dv)
    drain_writes((_CHUNKS - 1) % 3)


def kernel(words, classes, noun_table, class_table, special_emb):
    words_i = words.astype(jnp.int32)
    cls_i = classes.astype(jnp.int32)
    pe = jnp.asarray(_PE[:L_TOK])

    mesh = plsc.VectorSubcoreMesh(core_axis_name="c", subcore_axis_name="s")
    run = functools.partial(
        pl.kernel,
        mesh=mesh,
        compiler_params=pltpu.CompilerParams(needs_layout_passes=False),
        out_type=jax.ShapeDtypeStruct((B, MAX_LEN, D), jnp.float32),
        scratch_types=[
            pltpu.VMEM((_B_PER_W, L_TOK), jnp.int32),
            pltpu.VMEM((_B_PER_W, L_TOK), jnp.int32),
            pltpu.VMEM((L_TOK, D), jnp.float32),
            pltpu.VMEM((2, D), jnp.float32),
            pltpu.VMEM((1, D), jnp.float32),
            pltpu.VMEM((_OROWS, D), jnp.float32),
            pltpu.VMEM((_OROWS, D), jnp.float32),
            pltpu.VMEM((_OROWS, D), jnp.float32),
        ] + [pltpu.SemaphoreType.DMA] * 6,
    )(_sc_body)
    return run(words_i, cls_i, noun_table, class_table, special_emb, pe)
